# baseline (speedup 1.0000x reference)
import functools

import jax
import jax.numpy as jnp
from jax import lax
from jax.experimental import pallas as pl
from jax.experimental.pallas import tpu as pltpu
from jax.experimental.pallas import tpu_sc as plsc

N_NODES = 10000
HID = 128
PRE_LEN = 12
LANES = 16
N_PAD = 10240
NC = 2
NS = 16
NW = NC * NS
CHUNK = 128
EPW_CHUNKS = 81
E_PAD = NW * EPW_CHUNKS * CHUNK
ROWS_PER_TILE = N_PAD // NS

NB = 512
N_BLOCKS = N_PAD // NB


def _deg_body(col_hbm, ewb_hbm, zeros_hbm, deg_out,
              col_v, buf0, buf1, buf2, deg_sh, dsem, ssem):
    c = lax.axis_index("c")
    s = lax.axis_index("s")
    wid = c * NS + s
    bufs = (buf0, buf1, buf2)
    pltpu.sync_copy(col_hbm.at[wid], col_v)
    pltpu.sync_copy(zeros_hbm.at[pl.ds(s * ROWS_PER_TILE, ROWS_PER_TILE)],
                    deg_sh.at[pl.ds(s * ROWS_PER_TILE, ROWS_PER_TILE)])
    plsc.subcore_barrier()

    pltpu.sync_copy(ewb_hbm.at[wid, EPW_CHUNKS], buf2)
    pltpu.async_copy(buf2, deg_sh.at[col_v.at[EPW_CHUNKS]], ssem, add=True)
    pltpu.async_copy(buf2, deg_sh.at[col_v.at[EPW_CHUNKS]], ssem, add=True)
    pltpu.async_copy(ewb_hbm.at[wid, 0], buf0, dsem)

    def trip(p, carry):
        for b in range(3):
            j = 3 * p + b
            pltpu.make_async_copy(bufs[b], deg_sh.at[col_v.at[j]], ssem).wait()
            pltpu.async_copy(ewb_hbm.at[wid, j + 1], bufs[(b + 1) % 3], dsem)
            pltpu.make_async_copy(ewb_hbm.at[wid, j], bufs[b], dsem).wait()
            pltpu.async_copy(bufs[b], deg_sh.at[col_v.at[j]], ssem, add=True)
        return carry

    lax.fori_loop(0, EPW_CHUNKS // 3, trip, 0)
    pltpu.make_async_copy(buf0, deg_sh.at[col_v.at[0]], ssem).wait()
    pltpu.make_async_copy(buf1, deg_sh.at[col_v.at[0]], ssem).wait()
    pltpu.make_async_copy(ewb_hbm.at[wid, 0], buf0, dsem).wait()
    plsc.subcore_barrier()
    pltpu.sync_copy(deg_sh.at[pl.ds(s * ROWS_PER_TILE, ROWS_PER_TILE)],
                    deg_out.at[c, pl.ds(s * ROWS_PER_TILE, ROWS_PER_TILE)])


@functools.cache
def _make_deg_kernel():
    return pl.kernel(
        _deg_body,
        out_type=jax.ShapeDtypeStruct((NC, N_PAD, LANES), jnp.float32),
        mesh=plsc.VectorSubcoreMesh(core_axis_name="c", subcore_axis_name="s"),
        scratch_types=[
            pltpu.VMEM((EPW_CHUNKS + 1, CHUNK), jnp.int32),
            pltpu.VMEM((CHUNK, LANES), jnp.float32),
            pltpu.VMEM((CHUNK, LANES), jnp.float32),
            pltpu.VMEM((CHUNK, LANES), jnp.float32),
            pltpu.VMEM_SHARED((N_PAD, LANES), jnp.float32),
            pltpu.SemaphoreType.DMA,
            pltpu.SemaphoreType.DMA,
        ],
        compiler_params=pltpu.CompilerParams(use_tc_tiling_on_sc=False),
    )


def _prep_body(degmat_ref, x_ref, y_ref, dinv_ref):
    deg = degmat_ref[0] + degmat_ref[1] + 2.0
    dinv = lax.rsqrt(deg)
    dinv_ref[...] = dinv
    y_ref[...] = x_ref[...] * dinv


def _run_prep(degmat, x_r):
    R = N_PAD * LANES // 128
    return pl.pallas_call(
        _prep_body,
        out_shape=[jax.ShapeDtypeStruct((R, 128), jnp.float32),
                   jax.ShapeDtypeStruct((R, 128), jnp.float32)],
    )(degmat, x_r)


def _z_body(row_hbm, col_hbm, ewb_hbm, y_hbm, zeros_hbm, z_out,
            row_v, col_v,
            ybuf0, ybuf1, ybuf2, ewbuf0, ewbuf1, ewbuf2,
            zbuf0, zbuf1, zbuf2, z_sh, gsem, esem, ssem):
    c = lax.axis_index("c")
    s = lax.axis_index("s")
    wid = c * NS + s
    ybufs = (ybuf0, ybuf1, ybuf2)
    ewbufs = (ewbuf0, ewbuf1, ewbuf2)
    zbufs = (zbuf0, zbuf1, zbuf2)
    pltpu.sync_copy(row_hbm.at[wid], row_v)
    pltpu.sync_copy(col_hbm.at[wid], col_v)
    pltpu.sync_copy(zeros_hbm.at[pl.ds(s * ROWS_PER_TILE, ROWS_PER_TILE)],
                    z_sh.at[pl.ds(s * ROWS_PER_TILE, ROWS_PER_TILE)])
    plsc.subcore_barrier()

    pltpu.sync_copy(zeros_hbm.at[pl.ds(0, CHUNK)], zbuf2)
    pltpu.async_copy(zbuf2, z_sh.at[col_v.at[EPW_CHUNKS]], ssem, add=True)
    pltpu.async_copy(zbuf2, z_sh.at[col_v.at[EPW_CHUNKS]], ssem, add=True)
    pltpu.async_copy(y_hbm.at[row_v.at[0]], ybuf0, gsem)
    pltpu.async_copy(ewb_hbm.at[wid, 0], ewbuf0, esem)

    def trip(p, carry):
        for b in range(3):
            j = 3 * p + b
            nb = (b + 1) % 3
            pltpu.make_async_copy(zbufs[b], z_sh.at[col_v.at[j]], ssem).wait()
            pltpu.async_copy(y_hbm.at[row_v.at[j + 1]], ybufs[nb], gsem)
            pltpu.async_copy(ewb_hbm.at[wid, j + 1], ewbufs[nb], esem)
            pltpu.make_async_copy(y_hbm.at[row_v.at[j]], ybufs[b], gsem).wait()
            pltpu.make_async_copy(ewb_hbm.at[wid, j], ewbufs[b], esem).wait()
            for r in range(CHUNK):
                zbufs[b][r, :] = ybufs[b][r, :] * ewbufs[b][r, :]
            pltpu.async_copy(zbufs[b], z_sh.at[col_v.at[j]], ssem, add=True)
        return carry

    lax.fori_loop(0, EPW_CHUNKS // 3, trip, 0)
    pltpu.make_async_copy(zbuf0, z_sh.at[col_v.at[0]], ssem).wait()
    pltpu.make_async_copy(zbuf1, z_sh.at[col_v.at[0]], ssem).wait()
    pltpu.make_async_copy(y_hbm.at[row_v.at[0]], ybuf0, gsem).wait()
    pltpu.make_async_copy(ewb_hbm.at[wid, 0], ewbuf0, esem).wait()
    plsc.subcore_barrier()
    pltpu.sync_copy(z_sh.at[pl.ds(s * ROWS_PER_TILE, ROWS_PER_TILE)],
                    z_out.at[c, pl.ds(s * ROWS_PER_TILE, ROWS_PER_TILE)])


@functools.cache
def _make_z_kernel():
    return pl.kernel(
        _z_body,
        out_type=jax.ShapeDtypeStruct((NC, N_PAD, LANES), jnp.float32),
        mesh=plsc.VectorSubcoreMesh(core_axis_name="c", subcore_axis_name="s"),
        scratch_types=(
            [pltpu.VMEM((EPW_CHUNKS + 1, CHUNK), jnp.int32)] * 2
            + [pltpu.VMEM((CHUNK, LANES), jnp.float32)] * 9
            + [pltpu.VMEM_SHARED((N_PAD, LANES), jnp.float32),
               pltpu.SemaphoreType.DMA,
               pltpu.SemaphoreType.DMA,
               pltpu.SemaphoreType.DMA]
        ),
        compiler_params=pltpu.CompilerParams(use_tc_tiling_on_sc=False),
    )


def _gru_body(z_ref, y_ref, dinv_ref, wc_ref, bc_ref, w1_ref, b1_ref,
              w2_ref, b2_ref, out_ref):
    dinv = dinv_ref[...]
    s_all = dinv * (z_ref[0] + z_ref[1] + 2.0 * y_ref[...])
    wc = wc_ref[...]
    bc = bc_ref[...]
    b1 = b1_ref[...]
    b2 = b2_ref[...]
    w1 = w1_ref[...]
    w2 = w2_ref[...]
    h = jnp.zeros((NB, HID), jnp.float32)
    for t in range(PRE_LEN):
        st = s_all[:, t:t + 1]
        f = jax.nn.sigmoid(st * wc + bc)
        cat1 = jnp.concatenate([f, h], axis=1)
        ru = jax.nn.sigmoid(
            jnp.dot(cat1, w1, preferred_element_type=jnp.float32) + b1)
        r = ru[:, :HID]
        u = ru[:, HID:]
        cat2 = jnp.concatenate([f, r * h], axis=1)
        cnew = jnp.tanh(
            jnp.dot(cat2, w2, preferred_element_type=jnp.float32) + b2)
        h = u * h + (1.0 - u) * cnew
    out_ref[...] = h


def _run_gru(zmat, y2, dinv2, Wc, bc, W1, b1, W2, b2):
    grid = (N_BLOCKS,)
    return pl.pallas_call(
        _gru_body,
        grid=grid,
        in_specs=[
            pl.BlockSpec((NC, NB, LANES), lambda i: (0, i, 0)),
            pl.BlockSpec((NB, LANES), lambda i: (i, 0)),
            pl.BlockSpec((NB, LANES), lambda i: (i, 0)),
            pl.BlockSpec((1, HID), lambda i: (0, 0)),
            pl.BlockSpec((1, HID), lambda i: (0, 0)),
            pl.BlockSpec((2 * HID, 2 * HID), lambda i: (0, 0)),
            pl.BlockSpec((1, 2 * HID), lambda i: (0, 0)),
            pl.BlockSpec((2 * HID, HID), lambda i: (0, 0)),
            pl.BlockSpec((1, HID), lambda i: (0, 0)),
        ],
        out_specs=pl.BlockSpec((NB, HID), lambda i: (i, 0)),
        out_shape=jax.ShapeDtypeStruct((N_PAD, HID), jnp.float32),
    )(zmat, y2, dinv2, Wc, bc, W1, b1, W2, b2)


def kernel(x, edge_index, edge_weight, Wc, bc, W1, b1, W2, b2):
    E = edge_weight.shape[0]
    row = edge_index[0].astype(jnp.int32)
    col = edge_index[1].astype(jnp.int32)
    ew = edge_weight.astype(jnp.float32)

    pad_e = E_PAD - E
    row3 = jnp.pad(jnp.pad(row, (0, pad_e)).reshape(NW, EPW_CHUNKS, CHUNK),
                   ((0, 0), (0, 1), (0, 0)))
    col3 = jnp.pad(jnp.pad(col, (0, pad_e)).reshape(NW, EPW_CHUNKS, CHUNK),
                   ((0, 0), (0, 1), (0, 0)))
    ewp = jnp.pad(ew, (0, pad_e))
    ewb = jnp.pad(
        jnp.broadcast_to(ewp[:, None], (E_PAD, LANES)).reshape(
            NW, EPW_CHUNKS, CHUNK, LANES),
        ((0, 0), (0, 1), (0, 0), (0, 0)))

    x_pad = jnp.pad(x, ((0, N_PAD - N_NODES), (0, LANES - PRE_LEN)))
    zeros_pad = jnp.zeros((N_PAD, LANES), jnp.float32)

    degmat = _make_deg_kernel()(col3, ewb, zeros_pad)

    R = N_PAD * LANES // 128
    y_r, dinv_r = _run_prep(degmat.reshape(NC, R, 128), x_pad.reshape(R, 128))
    y2 = y_r.reshape(N_PAD, LANES)
    dinv2 = dinv_r.reshape(N_PAD, LANES)

    zmat = _make_z_kernel()(row3, col3, ewb, y2, zeros_pad)

    h = _run_gru(zmat, y2, dinv2, Wc, bc.reshape(1, HID), W1,
                 b1.reshape(1, 2 * HID), W2, b2.reshape(1, HID))
    return h[:N_NODES]


# speedup vs baseline: 2.1427x; 2.1427x over previous
import functools

import jax
import jax.numpy as jnp
from jax import lax
from jax.experimental import pallas as pl
from jax.experimental.pallas import tpu as pltpu
from jax.experimental.pallas import tpu_sc as plsc

N_NODES = 10000
HID = 128
PRE_LEN = 12
LANES = 16
N_PAD = 10240
NC = 2
NS = 16
NW = NC * NS
CHUNK = 128
EPW_CHUNKS = 81
E_PAD = NW * EPW_CHUNKS * CHUNK
ROWS_PER_TILE = N_PAD // NS

NB = 512
N_BLOCKS = N_PAD // NB


def _splat16(v):
    return jnp.zeros((LANES,), jnp.int32) + v


def _deg_body(col_hbm, ew_hbm, zeros_hbm, deg_out,
              col_v, ew_v, buf0, buf1, buf2, deg_sh, ssem):
    c = lax.axis_index("c")
    s = lax.axis_index("s")
    wid = c * NS + s
    bufs = (buf0, buf1, buf2)
    pltpu.sync_copy(col_hbm.at[wid], col_v)
    pltpu.sync_copy(ew_hbm.at[wid], ew_v)
    pltpu.sync_copy(zeros_hbm.at[pl.ds(s * ROWS_PER_TILE, ROWS_PER_TILE)],
                    deg_sh.at[pl.ds(s * ROWS_PER_TILE, ROWS_PER_TILE)])
    plsc.subcore_barrier()

    for r in range(CHUNK):
        buf2[r, :] = jnp.zeros((LANES,), jnp.float32)
    pltpu.async_copy(buf2, deg_sh.at[col_v.at[EPW_CHUNKS]], ssem, add=True)
    pltpu.async_copy(buf2, deg_sh.at[col_v.at[EPW_CHUNKS]], ssem, add=True)

    def trip(p, carry):
        for b in range(3):
            j = 3 * p + b
            pltpu.make_async_copy(bufs[b], deg_sh.at[col_v.at[j]], ssem).wait()
            js = _splat16(j)
            for r in range(CHUNK):
                w = plsc.load_gather(ew_v, [js, _splat16(r)])
                bufs[b][r, :] = w
            pltpu.async_copy(bufs[b], deg_sh.at[col_v.at[j]], ssem, add=True)
        return carry

    lax.fori_loop(0, EPW_CHUNKS // 3, trip, 0)
    pltpu.make_async_copy(buf0, deg_sh.at[col_v.at[0]], ssem).wait()
    pltpu.make_async_copy(buf1, deg_sh.at[col_v.at[0]], ssem).wait()
    plsc.subcore_barrier()
    pltpu.sync_copy(deg_sh.at[pl.ds(s * ROWS_PER_TILE, ROWS_PER_TILE)],
                    deg_out.at[c, pl.ds(s * ROWS_PER_TILE, ROWS_PER_TILE)])


@functools.cache
def _make_deg_kernel():
    return pl.kernel(
        _deg_body,
        out_type=jax.ShapeDtypeStruct((NC, N_PAD, LANES), jnp.float32),
        mesh=plsc.VectorSubcoreMesh(core_axis_name="c", subcore_axis_name="s"),
        scratch_types=[
            pltpu.VMEM((EPW_CHUNKS + 1, CHUNK), jnp.int32),
            pltpu.VMEM((EPW_CHUNKS + 1, CHUNK), jnp.float32),
            pltpu.VMEM((CHUNK, LANES), jnp.float32),
            pltpu.VMEM((CHUNK, LANES), jnp.float32),
            pltpu.VMEM((CHUNK, LANES), jnp.float32),
            pltpu.VMEM_SHARED((N_PAD, LANES), jnp.float32),
            pltpu.SemaphoreType.DMA,
        ],
        compiler_params=pltpu.CompilerParams(use_tc_tiling_on_sc=False, needs_layout_passes=False),
    )


def _prep_body(degmat_ref, x_ref, y_ref, dinv_ref):
    deg = degmat_ref[0] + degmat_ref[1] + 2.0
    dinv = lax.rsqrt(deg)
    dinv_ref[...] = dinv
    y_ref[...] = x_ref[...] * dinv


def _run_prep(degmat, x_r):
    R = N_PAD * LANES // 128
    return pl.pallas_call(
        _prep_body,
        out_shape=[jax.ShapeDtypeStruct((R, 128), jnp.float32),
                   jax.ShapeDtypeStruct((R, 128), jnp.float32)],
    )(degmat, x_r)


def _z_body(row_hbm, col_hbm, ew_hbm, y_hbm, zeros_hbm, z_out,
            row_v, col_v, ew_v,
            ybuf0, ybuf1, ybuf2, zbuf0, zbuf1, zbuf2,
            y_sh, z_sh, gsem, ssem):
    c = lax.axis_index("c")
    s = lax.axis_index("s")
    wid = c * NS + s
    ybufs = (ybuf0, ybuf1, ybuf2)
    zbufs = (zbuf0, zbuf1, zbuf2)
    pltpu.sync_copy(row_hbm.at[wid], row_v)
    pltpu.sync_copy(col_hbm.at[wid], col_v)
    pltpu.sync_copy(ew_hbm.at[wid], ew_v)
    pltpu.sync_copy(zeros_hbm.at[pl.ds(s * ROWS_PER_TILE, ROWS_PER_TILE)],
                    z_sh.at[pl.ds(s * ROWS_PER_TILE, ROWS_PER_TILE)])
    pltpu.sync_copy(y_hbm.at[pl.ds(s * ROWS_PER_TILE, ROWS_PER_TILE)],
                    y_sh.at[pl.ds(s * ROWS_PER_TILE, ROWS_PER_TILE)])
    plsc.subcore_barrier()

    for r in range(CHUNK):
        zbuf2[r, :] = jnp.zeros((LANES,), jnp.float32)
    pltpu.async_copy(zbuf2, z_sh.at[col_v.at[EPW_CHUNKS]], ssem, add=True)
    pltpu.async_copy(zbuf2, z_sh.at[col_v.at[EPW_CHUNKS]], ssem, add=True)
    pltpu.async_copy(y_sh.at[row_v.at[0]], ybuf0, gsem)

    def trip(p, carry):
        for b in range(3):
            j = 3 * p + b
            nb = (b + 1) % 3
            pltpu.make_async_copy(zbufs[b], z_sh.at[col_v.at[j]], ssem).wait()
            pltpu.async_copy(y_sh.at[row_v.at[j + 1]], ybufs[nb], gsem)
            pltpu.make_async_copy(y_sh.at[row_v.at[j]], ybufs[b], gsem).wait()
            js = _splat16(j)
            for r in range(CHUNK):
                w = plsc.load_gather(ew_v, [js, _splat16(r)])
                zbufs[b][r, :] = ybufs[b][r, :] * w
            pltpu.async_copy(zbufs[b], z_sh.at[col_v.at[j]], ssem, add=True)
        return carry

    lax.fori_loop(0, EPW_CHUNKS // 3, trip, 0)
    pltpu.make_async_copy(zbuf0, z_sh.at[col_v.at[0]], ssem).wait()
    pltpu.make_async_copy(zbuf1, z_sh.at[col_v.at[0]], ssem).wait()
    pltpu.make_async_copy(y_sh.at[row_v.at[0]], ybuf0, gsem).wait()
    plsc.subcore_barrier()
    pltpu.sync_copy(z_sh.at[pl.ds(s * ROWS_PER_TILE, ROWS_PER_TILE)],
                    z_out.at[c, pl.ds(s * ROWS_PER_TILE, ROWS_PER_TILE)])


@functools.cache
def _make_z_kernel():
    return pl.kernel(
        _z_body,
        out_type=jax.ShapeDtypeStruct((NC, N_PAD, LANES), jnp.float32),
        mesh=plsc.VectorSubcoreMesh(core_axis_name="c", subcore_axis_name="s"),
        scratch_types=(
            [pltpu.VMEM((EPW_CHUNKS + 1, CHUNK), jnp.int32)] * 2
            + [pltpu.VMEM((EPW_CHUNKS + 1, CHUNK), jnp.float32)]
            + [pltpu.VMEM((CHUNK, LANES), jnp.float32)] * 6
            + [pltpu.VMEM_SHARED((N_PAD, LANES), jnp.float32)] * 2
            + [pltpu.SemaphoreType.DMA, pltpu.SemaphoreType.DMA]
        ),
        compiler_params=pltpu.CompilerParams(use_tc_tiling_on_sc=False, needs_layout_passes=False),
    )


def _gru_body(z_ref, y_ref, dinv_ref, wc_ref, bc_ref, w1_ref, b1_ref,
              w2_ref, b2_ref, out_ref):
    dinv = dinv_ref[...]
    s_all = dinv * (z_ref[0] + z_ref[1] + 2.0 * y_ref[...])
    wc = wc_ref[...]
    bc = bc_ref[...]
    b1 = b1_ref[...]
    b2 = b2_ref[...]
    w1 = w1_ref[...]
    w2 = w2_ref[...]
    h = jnp.zeros((NB, HID), jnp.float32)
    for t in range(PRE_LEN):
        st = s_all[:, t:t + 1]
        f = jax.nn.sigmoid(st * wc + bc)
        cat1 = jnp.concatenate([f, h], axis=1)
        ru = jax.nn.sigmoid(
            jnp.dot(cat1, w1, preferred_element_type=jnp.float32) + b1)
        r = ru[:, :HID]
        u = ru[:, HID:]
        cat2 = jnp.concatenate([f, r * h], axis=1)
        cnew = jnp.tanh(
            jnp.dot(cat2, w2, preferred_element_type=jnp.float32) + b2)
        h = u * h + (1.0 - u) * cnew
    out_ref[...] = h


def _run_gru(zmat, y2, dinv2, Wc, bc, W1, b1, W2, b2):
    grid = (N_BLOCKS,)
    return pl.pallas_call(
        _gru_body,
        grid=grid,
        in_specs=[
            pl.BlockSpec((NC, NB, LANES), lambda i: (0, i, 0)),
            pl.BlockSpec((NB, LANES), lambda i: (i, 0)),
            pl.BlockSpec((NB, LANES), lambda i: (i, 0)),
            pl.BlockSpec((1, HID), lambda i: (0, 0)),
            pl.BlockSpec((1, HID), lambda i: (0, 0)),
            pl.BlockSpec((2 * HID, 2 * HID), lambda i: (0, 0)),
            pl.BlockSpec((1, 2 * HID), lambda i: (0, 0)),
            pl.BlockSpec((2 * HID, HID), lambda i: (0, 0)),
            pl.BlockSpec((1, HID), lambda i: (0, 0)),
        ],
        out_specs=pl.BlockSpec((NB, HID), lambda i: (i, 0)),
        out_shape=jax.ShapeDtypeStruct((N_PAD, HID), jnp.float32),
    )(zmat, y2, dinv2, Wc, bc, W1, b1, W2, b2)


def kernel(x, edge_index, edge_weight, Wc, bc, W1, b1, W2, b2):
    E = edge_weight.shape[0]
    row = edge_index[0].astype(jnp.int32)
    col = edge_index[1].astype(jnp.int32)
    ew = edge_weight.astype(jnp.float32)

    pad_e = E_PAD - E
    row3 = jnp.pad(jnp.pad(row, (0, pad_e)).reshape(NW, EPW_CHUNKS, CHUNK),
                   ((0, 0), (0, 1), (0, 0)))
    col3 = jnp.pad(jnp.pad(col, (0, pad_e)).reshape(NW, EPW_CHUNKS, CHUNK),
                   ((0, 0), (0, 1), (0, 0)))
    ew3 = jnp.pad(jnp.pad(ew, (0, pad_e)).reshape(NW, EPW_CHUNKS, CHUNK),
                  ((0, 0), (0, 1), (0, 0)))

    x_pad = jnp.pad(x, ((0, N_PAD - N_NODES), (0, LANES - PRE_LEN)))
    zeros_pad = jnp.zeros((N_PAD, LANES), jnp.float32)

    degmat = _make_deg_kernel()(col3, ew3, zeros_pad)

    R = N_PAD * LANES // 128
    y_r, dinv_r = _run_prep(degmat.reshape(NC, R, 128), x_pad.reshape(R, 128))
    y2 = y_r.reshape(N_PAD, LANES)
    dinv2 = dinv_r.reshape(N_PAD, LANES)

    zmat = _make_z_kernel()(row3, col3, ew3, y2, zeros_pad)

    h = _run_gru(zmat, y2, dinv2, Wc, bc.reshape(1, HID), W1,
                 b1.reshape(1, 2 * HID), W2, b2.reshape(1, HID))
    return h[:N_NODES]


# speedup vs baseline: 2.1553x; 1.0059x over previous
import functools

import jax
import jax.numpy as jnp
from jax import lax
from jax.experimental import pallas as pl
from jax.experimental.pallas import tpu as pltpu
from jax.experimental.pallas import tpu_sc as plsc

N_NODES = 10000
HID = 128
PRE_LEN = 12
LANES = 16
N_PAD = 10240
NC = 2
NS = 16
NW = NC * NS
CHUNK = 128
EPW_CHUNKS = 81
E_PAD = NW * EPW_CHUNKS * CHUNK
ROWS_PER_TILE = N_PAD // NS

NB = 1024
N_BLOCKS = N_PAD // NB


def _splat16(v):
    return jnp.zeros((LANES,), jnp.int32) + v


def _deg_body(col_hbm, ew_hbm, zeros_hbm, deg_out,
              col_v, ew_v, buf0, buf1, buf2, deg_sh, ssem):
    c = lax.axis_index("c")
    s = lax.axis_index("s")
    wid = c * NS + s
    bufs = (buf0, buf1, buf2)
    pltpu.sync_copy(col_hbm.at[wid], col_v)
    pltpu.sync_copy(ew_hbm.at[wid], ew_v)
    pltpu.sync_copy(zeros_hbm.at[pl.ds(s * ROWS_PER_TILE, ROWS_PER_TILE)],
                    deg_sh.at[pl.ds(s * ROWS_PER_TILE, ROWS_PER_TILE)])
    plsc.subcore_barrier()

    for r in range(CHUNK):
        buf2[r, :] = jnp.zeros((LANES,), jnp.float32)
    pltpu.async_copy(buf2, deg_sh.at[col_v.at[EPW_CHUNKS]], ssem, add=True)
    pltpu.async_copy(buf2, deg_sh.at[col_v.at[EPW_CHUNKS]], ssem, add=True)

    def trip(p, carry):
        for b in range(3):
            j = 3 * p + b
            pltpu.make_async_copy(bufs[b], deg_sh.at[col_v.at[j]], ssem).wait()
            js = _splat16(j)
            for r in range(CHUNK):
                w = plsc.load_gather(ew_v, [js, _splat16(r)])
                bufs[b][r, :] = w
            pltpu.async_copy(bufs[b], deg_sh.at[col_v.at[j]], ssem, add=True)
        return carry

    lax.fori_loop(0, EPW_CHUNKS // 3, trip, 0)
    pltpu.make_async_copy(buf0, deg_sh.at[col_v.at[0]], ssem).wait()
    pltpu.make_async_copy(buf1, deg_sh.at[col_v.at[0]], ssem).wait()
    plsc.subcore_barrier()
    pltpu.sync_copy(deg_sh.at[pl.ds(s * ROWS_PER_TILE, ROWS_PER_TILE)],
                    deg_out.at[c, pl.ds(s * ROWS_PER_TILE, ROWS_PER_TILE)])


@functools.cache
def _make_deg_kernel():
    return pl.kernel(
        _deg_body,
        out_type=jax.ShapeDtypeStruct((NC, N_PAD, LANES), jnp.float32),
        mesh=plsc.VectorSubcoreMesh(core_axis_name="c", subcore_axis_name="s"),
        scratch_types=[
            pltpu.VMEM((EPW_CHUNKS + 1, CHUNK), jnp.int32),
            pltpu.VMEM((EPW_CHUNKS + 1, CHUNK), jnp.float32),
            pltpu.VMEM((CHUNK, LANES), jnp.float32),
            pltpu.VMEM((CHUNK, LANES), jnp.float32),
            pltpu.VMEM((CHUNK, LANES), jnp.float32),
            pltpu.VMEM_SHARED((N_PAD, LANES), jnp.float32),
            pltpu.SemaphoreType.DMA,
        ],
        compiler_params=pltpu.CompilerParams(use_tc_tiling_on_sc=False, needs_layout_passes=False),
    )


def _prep_body(degmat_ref, x_ref, y_ref, dinv_ref):
    deg = degmat_ref[0] + degmat_ref[1] + 2.0
    dinv = lax.rsqrt(deg)
    dinv_ref[...] = dinv
    y_ref[...] = x_ref[...] * dinv


def _run_prep(degmat, x_pad):
    return pl.pallas_call(
        _prep_body,
        grid=(N_BLOCKS,),
        in_specs=[
            pl.BlockSpec((NC, NB, LANES), lambda i: (0, i, 0)),
            pl.BlockSpec((NB, LANES), lambda i: (i, 0)),
        ],
        out_specs=[pl.BlockSpec((NB, LANES), lambda i: (i, 0)),
                   pl.BlockSpec((NB, LANES), lambda i: (i, 0))],
        out_shape=[jax.ShapeDtypeStruct((N_PAD, LANES), jnp.float32),
                   jax.ShapeDtypeStruct((N_PAD, LANES), jnp.float32)],
    )(degmat, x_pad)


def _z_body(row_hbm, col_hbm, ew_hbm, y_hbm, zeros_hbm, z_out,
            row_v, col_v, ew_v,
            ybuf0, ybuf1, ybuf2, zbuf0, zbuf1, zbuf2,
            y_sh, z_sh, gsem, ssem):
    c = lax.axis_index("c")
    s = lax.axis_index("s")
    wid = c * NS + s
    ybufs = (ybuf0, ybuf1, ybuf2)
    zbufs = (zbuf0, zbuf1, zbuf2)
    pltpu.sync_copy(row_hbm.at[wid], row_v)
    pltpu.sync_copy(col_hbm.at[wid], col_v)
    pltpu.sync_copy(ew_hbm.at[wid], ew_v)
    pltpu.sync_copy(zeros_hbm.at[pl.ds(s * ROWS_PER_TILE, ROWS_PER_TILE)],
                    z_sh.at[pl.ds(s * ROWS_PER_TILE, ROWS_PER_TILE)])
    pltpu.sync_copy(y_hbm.at[pl.ds(s * ROWS_PER_TILE, ROWS_PER_TILE)],
                    y_sh.at[pl.ds(s * ROWS_PER_TILE, ROWS_PER_TILE)])
    plsc.subcore_barrier()

    for r in range(CHUNK):
        zbuf2[r, :] = jnp.zeros((LANES,), jnp.float32)
    pltpu.async_copy(zbuf2, z_sh.at[col_v.at[EPW_CHUNKS]], ssem, add=True)
    pltpu.async_copy(zbuf2, z_sh.at[col_v.at[EPW_CHUNKS]], ssem, add=True)
    pltpu.async_copy(y_sh.at[row_v.at[0]], ybuf0, gsem)

    def trip(p, carry):
        for b in range(3):
            j = 3 * p + b
            nb = (b + 1) % 3
            pltpu.make_async_copy(zbufs[b], z_sh.at[col_v.at[j]], ssem).wait()
            pltpu.async_copy(y_sh.at[row_v.at[j + 1]], ybufs[nb], gsem)
            pltpu.make_async_copy(y_sh.at[row_v.at[j]], ybufs[b], gsem).wait()
            js = _splat16(j)
            for r in range(CHUNK):
                w = plsc.load_gather(ew_v, [js, _splat16(r)])
                zbufs[b][r, :] = ybufs[b][r, :] * w
            pltpu.async_copy(zbufs[b], z_sh.at[col_v.at[j]], ssem, add=True)
        return carry

    lax.fori_loop(0, EPW_CHUNKS // 3, trip, 0)
    pltpu.make_async_copy(zbuf0, z_sh.at[col_v.at[0]], ssem).wait()
    pltpu.make_async_copy(zbuf1, z_sh.at[col_v.at[0]], ssem).wait()
    pltpu.make_async_copy(y_sh.at[row_v.at[0]], ybuf0, gsem).wait()
    plsc.subcore_barrier()
    pltpu.sync_copy(z_sh.at[pl.ds(s * ROWS_PER_TILE, ROWS_PER_TILE)],
                    z_out.at[c, pl.ds(s * ROWS_PER_TILE, ROWS_PER_TILE)])


@functools.cache
def _make_z_kernel():
    return pl.kernel(
        _z_body,
        out_type=jax.ShapeDtypeStruct((NC, N_PAD, LANES), jnp.float32),
        mesh=plsc.VectorSubcoreMesh(core_axis_name="c", subcore_axis_name="s"),
        scratch_types=(
            [pltpu.VMEM((EPW_CHUNKS + 1, CHUNK), jnp.int32)] * 2
            + [pltpu.VMEM((EPW_CHUNKS + 1, CHUNK), jnp.float32)]
            + [pltpu.VMEM((CHUNK, LANES), jnp.float32)] * 6
            + [pltpu.VMEM_SHARED((N_PAD, LANES), jnp.float32)] * 2
            + [pltpu.SemaphoreType.DMA, pltpu.SemaphoreType.DMA]
        ),
        compiler_params=pltpu.CompilerParams(use_tc_tiling_on_sc=False, needs_layout_passes=False),
    )


def _gru_body(z_ref, y_ref, dinv_ref, wc_ref, bc_ref, w1_ref, b1_ref,
              w2_ref, b2_ref, out_ref):
    dinv = dinv_ref[...]
    s_all = dinv * (z_ref[0] + z_ref[1] + 2.0 * y_ref[...])
    wc = wc_ref[...]
    bc = bc_ref[...]
    b1 = b1_ref[...]
    b2 = b2_ref[...]
    w1 = w1_ref[...]
    w2 = w2_ref[...]
    h = jnp.zeros((NB, HID), jnp.float32)
    for t in range(PRE_LEN):
        st = s_all[:, t:t + 1]
        f = jax.nn.sigmoid(st * wc + bc)
        cat1 = jnp.concatenate([f, h], axis=1)
        ru = jax.nn.sigmoid(
            jnp.dot(cat1, w1, preferred_element_type=jnp.float32) + b1)
        r = ru[:, :HID]
        u = ru[:, HID:]
        cat2 = jnp.concatenate([f, r * h], axis=1)
        cnew = jnp.tanh(
            jnp.dot(cat2, w2, preferred_element_type=jnp.float32) + b2)
        h = u * h + (1.0 - u) * cnew
    out_ref[...] = h


def _run_gru(zmat, y2, dinv2, Wc, bc, W1, b1, W2, b2):
    grid = (N_BLOCKS,)
    return pl.pallas_call(
        _gru_body,
        grid=grid,
        in_specs=[
            pl.BlockSpec((NC, NB, LANES), lambda i: (0, i, 0)),
            pl.BlockSpec((NB, LANES), lambda i: (i, 0)),
            pl.BlockSpec((NB, LANES), lambda i: (i, 0)),
            pl.BlockSpec((1, HID), lambda i: (0, 0)),
            pl.BlockSpec((1, HID), lambda i: (0, 0)),
            pl.BlockSpec((2 * HID, 2 * HID), lambda i: (0, 0)),
            pl.BlockSpec((1, 2 * HID), lambda i: (0, 0)),
            pl.BlockSpec((2 * HID, HID), lambda i: (0, 0)),
            pl.BlockSpec((1, HID), lambda i: (0, 0)),
        ],
        out_specs=pl.BlockSpec((NB, HID), lambda i: (i, 0)),
        out_shape=jax.ShapeDtypeStruct((N_PAD, HID), jnp.float32),
    )(zmat, y2, dinv2, Wc, bc, W1, b1, W2, b2)


def kernel(x, edge_index, edge_weight, Wc, bc, W1, b1, W2, b2):
    E = edge_weight.shape[0]
    row = edge_index[0].astype(jnp.int32)
    col = edge_index[1].astype(jnp.int32)
    ew = edge_weight.astype(jnp.float32)

    pad_e = E_PAD - E
    row3 = jnp.pad(jnp.pad(row, (0, pad_e)).reshape(NW, EPW_CHUNKS, CHUNK),
                   ((0, 0), (0, 1), (0, 0)))
    col3 = jnp.pad(jnp.pad(col, (0, pad_e)).reshape(NW, EPW_CHUNKS, CHUNK),
                   ((0, 0), (0, 1), (0, 0)))
    ew3 = jnp.pad(jnp.pad(ew, (0, pad_e)).reshape(NW, EPW_CHUNKS, CHUNK),
                  ((0, 0), (0, 1), (0, 0)))

    x_pad = jnp.pad(x, ((0, N_PAD - N_NODES), (0, LANES - PRE_LEN)))
    zeros_pad = jnp.zeros((N_PAD, LANES), jnp.float32)

    degmat = _make_deg_kernel()(col3, ew3, zeros_pad)

    y2, dinv2 = _run_prep(degmat, x_pad)

    zmat = _make_z_kernel()(row3, col3, ew3, y2, zeros_pad)

    h = _run_gru(zmat, y2, dinv2, Wc, bc.reshape(1, HID), W1,
                 b1.reshape(1, 2 * HID), W2, b2.reshape(1, HID))
    return h[:N_NODES]


# speedup vs baseline: 2.2240x; 1.0318x over previous
import functools

import jax
import jax.numpy as jnp
from jax import lax
from jax.experimental import pallas as pl
from jax.experimental.pallas import tpu as pltpu
from jax.experimental.pallas import tpu_sc as plsc

N_NODES = 10000
HID = 128
PRE_LEN = 12
LANES = 16
N_PAD = 10240
NC = 2
NS = 16
NW = NC * NS
CHUNK = 128
EPW_CHUNKS = 81
E_PAD = NW * EPW_CHUNKS * CHUNK
ROWS_PER_TILE = N_PAD // NS

NB = 1024
N_BLOCKS = N_PAD // NB


def _splat16(v):
    return jnp.zeros((LANES,), jnp.int32) + v


def _deg_body(ei_hbm, ew_hbm, zeros_hbm, deg_out,
              col_v, ew_v, buf0, buf1, buf2, deg_sh, ssem):
    c = lax.axis_index("c")
    s = lax.axis_index("s")
    wid = c * NS + s
    bufs = (buf0, buf1, buf2)
    pltpu.sync_copy(ei_hbm.at[1, wid], col_v)
    pltpu.sync_copy(ew_hbm.at[wid], ew_v)
    pltpu.sync_copy(zeros_hbm.at[pl.ds(s * ROWS_PER_TILE, ROWS_PER_TILE)],
                    deg_sh.at[pl.ds(s * ROWS_PER_TILE, ROWS_PER_TILE)])
    plsc.subcore_barrier()

    for r in range(CHUNK):
        buf2[r, :] = jnp.zeros((LANES,), jnp.float32)
    pltpu.async_copy(buf2, deg_sh.at[col_v.at[EPW_CHUNKS]], ssem, add=True)
    pltpu.async_copy(buf2, deg_sh.at[col_v.at[EPW_CHUNKS]], ssem, add=True)

    def trip(p, carry):
        for b in range(3):
            j = 3 * p + b
            pltpu.make_async_copy(bufs[b], deg_sh.at[col_v.at[j]], ssem).wait()
            js = _splat16(j)
            for r in range(CHUNK):
                w = plsc.load_gather(ew_v, [js, _splat16(r)])
                bufs[b][r, :] = w
            pltpu.async_copy(bufs[b], deg_sh.at[col_v.at[j]], ssem, add=True)
        return carry

    lax.fori_loop(0, EPW_CHUNKS // 3, trip, 0)
    pltpu.make_async_copy(buf0, deg_sh.at[col_v.at[0]], ssem).wait()
    pltpu.make_async_copy(buf1, deg_sh.at[col_v.at[0]], ssem).wait()
    plsc.subcore_barrier()
    pltpu.sync_copy(deg_sh.at[pl.ds(s * ROWS_PER_TILE, ROWS_PER_TILE)],
                    deg_out.at[c, pl.ds(s * ROWS_PER_TILE, ROWS_PER_TILE)])


@functools.cache
def _make_deg_kernel():
    return pl.kernel(
        _deg_body,
        out_type=jax.ShapeDtypeStruct((NC, N_PAD, LANES), jnp.float32),
        mesh=plsc.VectorSubcoreMesh(core_axis_name="c", subcore_axis_name="s"),
        scratch_types=[
            pltpu.VMEM((EPW_CHUNKS + 1, CHUNK), jnp.int32),
            pltpu.VMEM((EPW_CHUNKS + 1, CHUNK), jnp.float32),
            pltpu.VMEM((CHUNK, LANES), jnp.float32),
            pltpu.VMEM((CHUNK, LANES), jnp.float32),
            pltpu.VMEM((CHUNK, LANES), jnp.float32),
            pltpu.VMEM_SHARED((N_PAD, LANES), jnp.float32),
            pltpu.SemaphoreType.DMA,
        ],
        compiler_params=pltpu.CompilerParams(use_tc_tiling_on_sc=False, needs_layout_passes=False),
    )


def _prep_body(degmat_ref, x_ref, y_ref, dinv_ref):
    deg = degmat_ref[0] + degmat_ref[1] + 2.0
    dinv = lax.rsqrt(deg)
    dinv_ref[...] = dinv
    y_ref[...] = x_ref[...] * dinv


def _run_prep(degmat, x_pad):
    NBP = 2048
    return pl.pallas_call(
        _prep_body,
        grid=(N_PAD // NBP,),
        in_specs=[
            pl.BlockSpec((NC, NBP, LANES), lambda i: (0, i, 0)),
            pl.BlockSpec((NBP, LANES), lambda i: (i, 0)),
        ],
        out_specs=[pl.BlockSpec((NBP, LANES), lambda i: (i, 0)),
                   pl.BlockSpec((NBP, LANES), lambda i: (i, 0))],
        out_shape=[jax.ShapeDtypeStruct((N_PAD, LANES), jnp.float32),
                   jax.ShapeDtypeStruct((N_PAD, LANES), jnp.float32)],
    )(degmat, x_pad)


def _z_body(ei_hbm, ew_hbm, y_hbm, zeros_hbm, z_out,
            row_v, col_v, ew_v,
            ybuf0, ybuf1, ybuf2, zbuf0, zbuf1, zbuf2,
            y_sh, z_sh, gsem, ssem):
    c = lax.axis_index("c")
    s = lax.axis_index("s")
    wid = c * NS + s
    ybufs = (ybuf0, ybuf1, ybuf2)
    zbufs = (zbuf0, zbuf1, zbuf2)
    pltpu.sync_copy(ei_hbm.at[0, wid], row_v)
    pltpu.sync_copy(ei_hbm.at[1, wid], col_v)
    pltpu.sync_copy(ew_hbm.at[wid], ew_v)
    pltpu.sync_copy(zeros_hbm.at[pl.ds(s * ROWS_PER_TILE, ROWS_PER_TILE)],
                    z_sh.at[pl.ds(s * ROWS_PER_TILE, ROWS_PER_TILE)])
    pltpu.sync_copy(y_hbm.at[pl.ds(s * ROWS_PER_TILE, ROWS_PER_TILE)],
                    y_sh.at[pl.ds(s * ROWS_PER_TILE, ROWS_PER_TILE)])
    plsc.subcore_barrier()

    for r in range(CHUNK):
        zbuf2[r, :] = jnp.zeros((LANES,), jnp.float32)
    pltpu.async_copy(zbuf2, z_sh.at[col_v.at[EPW_CHUNKS]], ssem, add=True)
    pltpu.async_copy(zbuf2, z_sh.at[col_v.at[EPW_CHUNKS]], ssem, add=True)
    pltpu.async_copy(y_sh.at[row_v.at[0]], ybuf0, gsem)

    def trip(p, carry):
        for b in range(3):
            j = 3 * p + b
            nb = (b + 1) % 3
            pltpu.async_copy(y_sh.at[row_v.at[j + 1]], ybufs[nb], gsem)
            pltpu.make_async_copy(zbufs[b], z_sh.at[col_v.at[j]], ssem).wait()
            pltpu.make_async_copy(y_sh.at[row_v.at[j]], ybufs[b], gsem).wait()
            js = _splat16(j)
            for r in range(CHUNK):
                w = plsc.load_gather(ew_v, [js, _splat16(r)])
                zbufs[b][r, :] = ybufs[b][r, :] * w
            pltpu.async_copy(zbufs[b], z_sh.at[col_v.at[j]], ssem, add=True)
        return carry

    lax.fori_loop(0, EPW_CHUNKS // 3, trip, 0)
    pltpu.make_async_copy(zbuf0, z_sh.at[col_v.at[0]], ssem).wait()
    pltpu.make_async_copy(zbuf1, z_sh.at[col_v.at[0]], ssem).wait()
    pltpu.make_async_copy(y_sh.at[row_v.at[0]], ybuf0, gsem).wait()
    plsc.subcore_barrier()
    pltpu.sync_copy(z_sh.at[pl.ds(s * ROWS_PER_TILE, ROWS_PER_TILE)],
                    z_out.at[c, pl.ds(s * ROWS_PER_TILE, ROWS_PER_TILE)])


@functools.cache
def _make_z_kernel():
    return pl.kernel(
        _z_body,
        out_type=jax.ShapeDtypeStruct((NC, N_PAD, LANES), jnp.float32),
        mesh=plsc.VectorSubcoreMesh(core_axis_name="c", subcore_axis_name="s"),
        scratch_types=(
            [pltpu.VMEM((EPW_CHUNKS + 1, CHUNK), jnp.int32)] * 2
            + [pltpu.VMEM((EPW_CHUNKS + 1, CHUNK), jnp.float32)]
            + [pltpu.VMEM((CHUNK, LANES), jnp.float32)] * 6
            + [pltpu.VMEM_SHARED((N_PAD, LANES), jnp.float32)] * 2
            + [pltpu.SemaphoreType.DMA, pltpu.SemaphoreType.DMA]
        ),
        compiler_params=pltpu.CompilerParams(use_tc_tiling_on_sc=False, needs_layout_passes=False),
    )


def _gru_body(z_ref, y_ref, dinv_ref, wc_ref, bc_ref, w1_ref, b1_ref,
              w2_ref, b2_ref, out_ref):
    dinv = dinv_ref[...]
    s_all = dinv * (z_ref[0] + z_ref[1] + 2.0 * y_ref[...])
    wc = wc_ref[...]
    bc = bc_ref[...]
    b1 = b1_ref[...]
    b2 = b2_ref[...]
    w1 = w1_ref[...]
    w2 = w2_ref[...]
    h = jnp.zeros((NB, HID), jnp.float32)
    for t in range(PRE_LEN):
        st = s_all[:, t:t + 1]
        f = jax.nn.sigmoid(st * wc + bc)
        cat1 = jnp.concatenate([f, h], axis=1)
        ru = jax.nn.sigmoid(
            jnp.dot(cat1, w1, preferred_element_type=jnp.float32) + b1)
        r = ru[:, :HID]
        u = ru[:, HID:]
        cat2 = jnp.concatenate([f, r * h], axis=1)
        cnew = jnp.tanh(
            jnp.dot(cat2, w2, preferred_element_type=jnp.float32) + b2)
        h = u * h + (1.0 - u) * cnew
    out_ref[...] = h


def _run_gru(zmat, y2, dinv2, Wc, bc, W1, b1, W2, b2):
    grid = (N_BLOCKS,)
    return pl.pallas_call(
        _gru_body,
        grid=grid,
        in_specs=[
            pl.BlockSpec((NC, NB, LANES), lambda i: (0, i, 0)),
            pl.BlockSpec((NB, LANES), lambda i: (i, 0)),
            pl.BlockSpec((NB, LANES), lambda i: (i, 0)),
            pl.BlockSpec((1, HID), lambda i: (0, 0)),
            pl.BlockSpec((1, HID), lambda i: (0, 0)),
            pl.BlockSpec((2 * HID, 2 * HID), lambda i: (0, 0)),
            pl.BlockSpec((1, 2 * HID), lambda i: (0, 0)),
            pl.BlockSpec((2 * HID, HID), lambda i: (0, 0)),
            pl.BlockSpec((1, HID), lambda i: (0, 0)),
        ],
        out_specs=pl.BlockSpec((NB, HID), lambda i: (i, 0)),
        out_shape=jax.ShapeDtypeStruct((N_PAD, HID), jnp.float32),
    )(zmat, y2, dinv2, Wc, bc, W1, b1, W2, b2)


def kernel(x, edge_index, edge_weight, Wc, bc, W1, b1, W2, b2):
    E = edge_weight.shape[0]
    row = edge_index[0].astype(jnp.int32)
    col = edge_index[1].astype(jnp.int32)
    ew = edge_weight.astype(jnp.float32)

    pad_e = E_PAD - E
    ei4 = jnp.pad(
        jnp.pad(edge_index.astype(jnp.int32), ((0, 0), (0, pad_e))).reshape(
            2, NW, EPW_CHUNKS, CHUNK),
        ((0, 0), (0, 0), (0, 1), (0, 0)))
    ew3 = jnp.pad(jnp.pad(ew, (0, pad_e)).reshape(NW, EPW_CHUNKS, CHUNK),
                  ((0, 0), (0, 1), (0, 0)))

    x_pad = jnp.pad(x, ((0, N_PAD - N_NODES), (0, LANES - PRE_LEN)))
    zeros_pad = jnp.zeros((N_PAD, LANES), jnp.float32)

    degmat = _make_deg_kernel()(ei4, ew3, zeros_pad)

    y2, dinv2 = _run_prep(degmat, x_pad)

    zmat = _make_z_kernel()(ei4, ew3, y2, zeros_pad)

    h = _run_gru(zmat, y2, dinv2, Wc, bc.reshape(1, HID), W1,
                 b1.reshape(1, 2 * HID), W2, b2.reshape(1, HID))
    return h[:N_NODES]


# speedup vs baseline: 2.4102x; 1.0837x over previous
import functools

import jax
import jax.numpy as jnp
from jax import lax
from jax.experimental import pallas as pl
from jax.experimental.pallas import tpu as pltpu
from jax.experimental.pallas import tpu_sc as plsc

N_NODES = 10000
HID = 128
PRE_LEN = 12
LANES = 16
N_PAD = 10240
NC = 2
NS = 16
NW = NC * NS
CHUNK = 128
EPW_CHUNKS = 81
E_PAD = NW * EPW_CHUNKS * CHUNK
ROWS_PER_TILE = N_PAD // NS

NB = 1024
N_BLOCKS = N_PAD // NB


def _splat16(v):
    return jnp.zeros((LANES,), jnp.int32) + v


DEG_ROWS = N_PAD // LANES


def _deg_body(ei_hbm, ew_hbm, zeros_hbm, deg_out,
              col_v, ew_v, degp_v, iidx_v, tmp_v, deg_sh, ssem):
    c = lax.axis_index("c")
    s = lax.axis_index("s")
    wid = c * NS + s
    pltpu.sync_copy(ei_hbm.at[1, wid], col_v)
    pltpu.sync_copy(ew_hbm.at[wid], ew_v)
    rpt = DEG_ROWS // NS
    pltpu.sync_copy(zeros_hbm.at[pl.ds(s * rpt, rpt)],
                    deg_sh.at[pl.ds(s * rpt, rpt)])

    def zb(i, carry):
        degp_v[i, :] = jnp.zeros((LANES,), jnp.float32)
        return carry
    lax.fori_loop(0, DEG_ROWS, zb, 0)
    io = jnp.arange(LANES, dtype=jnp.int32)
    for q in range(DEG_ROWS // CHUNK):
        for k in range(CHUNK // LANES):
            iidx_v[q, pl.ds(k * LANES, LANES)] = io + (q * CHUNK + k * LANES)
    ids1 = jnp.minimum(io + 1, LANES - 1)
    plsc.subcore_barrier()

    def chunk(j, carry):
        for k in range(CHUNK // LANES):
            cols = col_v[j, pl.ds(k * LANES, LANES)]
            ws = ew_v[j, pl.ds(k * LANES, LANES)]
            sk, sv = plsc.sort_key_val(cols, ws)
            ps = plsc.cumsum(sv)
            tmp_v[k, :] = sk
            nextk = plsc.load_gather(tmp_v, [_splat16(k), ids1])
            runend = (sk != nextk) | (io == LANES - 1)
            m2 = runend & (io < LANES - 1)
            r1 = lax.shift_right_logical(sk, 4)
            l1 = lax.bitwise_and(sk, LANES - 1)
            plsc.addupdate_scatter(degp_v, [r1, l1], ps, mask=runend)
            r2 = lax.shift_right_logical(nextk, 4)
            l2 = lax.bitwise_and(nextk, LANES - 1)
            plsc.addupdate_scatter(degp_v, [r2, l2], -ps, mask=m2)
        return carry

    lax.fori_loop(0, EPW_CHUNKS, chunk, 0)

    for q in range(DEG_ROWS // CHUNK):
        pltpu.sync_copy(degp_v.at[pl.ds(q * CHUNK, CHUNK)],
                        deg_sh.at[iidx_v.at[q]], add=True)
    plsc.subcore_barrier()
    pltpu.sync_copy(deg_sh.at[pl.ds(s * rpt, rpt)],
                    deg_out.at[c, pl.ds(s * rpt, rpt)])


@functools.cache
def _make_deg_kernel():
    return pl.kernel(
        _deg_body,
        out_type=jax.ShapeDtypeStruct((NC, DEG_ROWS, LANES), jnp.float32),
        mesh=plsc.VectorSubcoreMesh(core_axis_name="c", subcore_axis_name="s"),
        scratch_types=[
            pltpu.VMEM((EPW_CHUNKS + 1, CHUNK), jnp.int32),
            pltpu.VMEM((EPW_CHUNKS + 1, CHUNK), jnp.float32),
            pltpu.VMEM((DEG_ROWS, LANES), jnp.float32),
            pltpu.VMEM((DEG_ROWS // CHUNK, CHUNK), jnp.int32),
            pltpu.VMEM((CHUNK // LANES, LANES), jnp.int32),
            pltpu.VMEM_SHARED((DEG_ROWS, LANES), jnp.float32),
            pltpu.SemaphoreType.DMA,
        ],
        compiler_params=pltpu.CompilerParams(use_tc_tiling_on_sc=False, needs_layout_passes=False),
    )


def _prep_body(degmat_ref, x_ref, y_ref, dinv_ref):
    deg = degmat_ref[0] + degmat_ref[1] + 2.0
    dinv = lax.rsqrt(deg)
    dinv_ref[...] = dinv
    y_ref[...] = x_ref[...] * dinv


def _run_prep(degmat, x_pad):
    NBP = 2048
    return pl.pallas_call(
        _prep_body,
        grid=(N_PAD // NBP,),
        in_specs=[
            pl.BlockSpec((NC, NBP, 1), lambda i: (0, i, 0)),
            pl.BlockSpec((NBP, LANES), lambda i: (i, 0)),
        ],
        out_specs=[pl.BlockSpec((NBP, LANES), lambda i: (i, 0)),
                   pl.BlockSpec((NBP, 1), lambda i: (i, 0))],
        out_shape=[jax.ShapeDtypeStruct((N_PAD, LANES), jnp.float32),
                   jax.ShapeDtypeStruct((N_PAD, 1), jnp.float32)],
    )(degmat, x_pad)


def _z_body(ei_hbm, ew_hbm, y_hbm, zeros_hbm, z_out,
            row_v, col_v, ew_v,
            ybuf0, ybuf1, ybuf2, zbuf0, zbuf1, zbuf2,
            y_sh, z_sh, gsem, ssem):
    c = lax.axis_index("c")
    s = lax.axis_index("s")
    wid = c * NS + s
    ybufs = (ybuf0, ybuf1, ybuf2)
    zbufs = (zbuf0, zbuf1, zbuf2)
    pltpu.sync_copy(ei_hbm.at[0, wid], row_v)
    pltpu.sync_copy(ei_hbm.at[1, wid], col_v)
    pltpu.sync_copy(ew_hbm.at[wid], ew_v)
    pltpu.sync_copy(zeros_hbm.at[pl.ds(s * ROWS_PER_TILE, ROWS_PER_TILE)],
                    z_sh.at[pl.ds(s * ROWS_PER_TILE, ROWS_PER_TILE)])
    pltpu.sync_copy(y_hbm.at[pl.ds(s * ROWS_PER_TILE, ROWS_PER_TILE)],
                    y_sh.at[pl.ds(s * ROWS_PER_TILE, ROWS_PER_TILE)])
    plsc.subcore_barrier()

    for r in range(CHUNK):
        zbuf2[r, :] = jnp.zeros((LANES,), jnp.float32)
    pltpu.async_copy(zbuf2, z_sh.at[col_v.at[EPW_CHUNKS]], ssem, add=True)
    pltpu.async_copy(zbuf2, z_sh.at[col_v.at[EPW_CHUNKS]], ssem, add=True)
    pltpu.async_copy(y_sh.at[row_v.at[0]], ybuf0, gsem)

    def trip(p, carry):
        for b in range(3):
            j = 3 * p + b
            nb = (b + 1) % 3
            pltpu.async_copy(y_sh.at[row_v.at[j + 1]], ybufs[nb], gsem)
            pltpu.make_async_copy(zbufs[b], z_sh.at[col_v.at[j]], ssem).wait()
            pltpu.make_async_copy(y_sh.at[row_v.at[j]], ybufs[b], gsem).wait()
            js = _splat16(j)
            for r in range(CHUNK):
                w = plsc.load_gather(ew_v, [js, _splat16(r)])
                zbufs[b][r, :] = ybufs[b][r, :] * w
            pltpu.async_copy(zbufs[b], z_sh.at[col_v.at[j]], ssem, add=True)
        return carry

    lax.fori_loop(0, EPW_CHUNKS // 3, trip, 0)
    pltpu.make_async_copy(zbuf0, z_sh.at[col_v.at[0]], ssem).wait()
    pltpu.make_async_copy(zbuf1, z_sh.at[col_v.at[0]], ssem).wait()
    pltpu.make_async_copy(y_sh.at[row_v.at[0]], ybuf0, gsem).wait()
    plsc.subcore_barrier()
    pltpu.sync_copy(z_sh.at[pl.ds(s * ROWS_PER_TILE, ROWS_PER_TILE)],
                    z_out.at[c, pl.ds(s * ROWS_PER_TILE, ROWS_PER_TILE)])


@functools.cache
def _make_z_kernel():
    return pl.kernel(
        _z_body,
        out_type=jax.ShapeDtypeStruct((NC, N_PAD, LANES), jnp.float32),
        mesh=plsc.VectorSubcoreMesh(core_axis_name="c", subcore_axis_name="s"),
        scratch_types=(
            [pltpu.VMEM((EPW_CHUNKS + 1, CHUNK), jnp.int32)] * 2
            + [pltpu.VMEM((EPW_CHUNKS + 1, CHUNK), jnp.float32)]
            + [pltpu.VMEM((CHUNK, LANES), jnp.float32)] * 6
            + [pltpu.VMEM_SHARED((N_PAD, LANES), jnp.float32)] * 2
            + [pltpu.SemaphoreType.DMA, pltpu.SemaphoreType.DMA]
        ),
        compiler_params=pltpu.CompilerParams(use_tc_tiling_on_sc=False, needs_layout_passes=False),
    )


def _gru_body(z_ref, y_ref, dinv_ref, wc_ref, bc_ref, w1_ref, b1_ref,
              w2_ref, b2_ref, out_ref):
    dinv = dinv_ref[...]
    s_all = dinv * (z_ref[0] + z_ref[1] + 2.0 * y_ref[...])
    wc = wc_ref[...]
    bc = bc_ref[...]
    b1 = b1_ref[...]
    b2 = b2_ref[...]
    w1 = w1_ref[...]
    w2 = w2_ref[...]
    h = jnp.zeros((NB, HID), jnp.float32)
    for t in range(PRE_LEN):
        st = s_all[:, t:t + 1]
        f = jax.nn.sigmoid(st * wc + bc)
        cat1 = jnp.concatenate([f, h], axis=1)
        ru = jax.nn.sigmoid(
            jnp.dot(cat1, w1, preferred_element_type=jnp.float32) + b1)
        r = ru[:, :HID]
        u = ru[:, HID:]
        cat2 = jnp.concatenate([f, r * h], axis=1)
        cnew = jnp.tanh(
            jnp.dot(cat2, w2, preferred_element_type=jnp.float32) + b2)
        h = u * h + (1.0 - u) * cnew
    out_ref[...] = h


def _run_gru(zmat, y2, dinv2, Wc, bc, W1, b1, W2, b2):
    grid = (N_BLOCKS,)
    return pl.pallas_call(
        _gru_body,
        grid=grid,
        in_specs=[
            pl.BlockSpec((NC, NB, LANES), lambda i: (0, i, 0)),
            pl.BlockSpec((NB, LANES), lambda i: (i, 0)),
            pl.BlockSpec((NB, 1), lambda i: (i, 0)),
            pl.BlockSpec((1, HID), lambda i: (0, 0)),
            pl.BlockSpec((1, HID), lambda i: (0, 0)),
            pl.BlockSpec((2 * HID, 2 * HID), lambda i: (0, 0)),
            pl.BlockSpec((1, 2 * HID), lambda i: (0, 0)),
            pl.BlockSpec((2 * HID, HID), lambda i: (0, 0)),
            pl.BlockSpec((1, HID), lambda i: (0, 0)),
        ],
        out_specs=pl.BlockSpec((NB, HID), lambda i: (i, 0)),
        out_shape=jax.ShapeDtypeStruct((N_PAD, HID), jnp.float32),
    )(zmat, y2, dinv2, Wc, bc, W1, b1, W2, b2)


def kernel(x, edge_index, edge_weight, Wc, bc, W1, b1, W2, b2):
    E = edge_weight.shape[0]
    row = edge_index[0].astype(jnp.int32)
    col = edge_index[1].astype(jnp.int32)
    ew = edge_weight.astype(jnp.float32)

    pad_e = E_PAD - E
    ei4 = jnp.pad(
        jnp.pad(edge_index.astype(jnp.int32), ((0, 0), (0, pad_e))).reshape(
            2, NW, EPW_CHUNKS, CHUNK),
        ((0, 0), (0, 0), (0, 1), (0, 0)))
    ew3 = jnp.pad(jnp.pad(ew, (0, pad_e)).reshape(NW, EPW_CHUNKS, CHUNK),
                  ((0, 0), (0, 1), (0, 0)))

    x_pad = jnp.pad(x, ((0, N_PAD - N_NODES), (0, LANES - PRE_LEN)))
    zeros_pad = jnp.zeros((N_PAD, LANES), jnp.float32)

    degmat = _make_deg_kernel()(ei4, ew3, zeros_pad)
    degmat = degmat.reshape(NC, N_PAD, 1)

    y2, dinv2 = _run_prep(degmat, x_pad)

    zmat = _make_z_kernel()(ei4, ew3, y2, zeros_pad)

    h = _run_gru(zmat, y2, dinv2, Wc, bc.reshape(1, HID), W1,
                 b1.reshape(1, 2 * HID), W2, b2.reshape(1, HID))
    return h[:N_NODES]


# speedup vs baseline: 2.4416x; 1.0130x over previous
import functools

import jax
import jax.numpy as jnp
from jax import lax
from jax.experimental import pallas as pl
from jax.experimental.pallas import tpu as pltpu
from jax.experimental.pallas import tpu_sc as plsc

N_NODES = 10000
HID = 128
PRE_LEN = 12
LANES = 16
N_PAD = 10240
NC = 2
NS = 16
NW = NC * NS
CHUNK = 128
EPW_CHUNKS = 81
E_PAD = NW * EPW_CHUNKS * CHUNK
ROWS_PER_TILE = N_PAD // NS

NB = 1024
N_BLOCKS = N_PAD // NB


def _splat16(v):
    return jnp.zeros((LANES,), jnp.int32) + v


DEG_ROWS = N_PAD // LANES


def _deg_body(ei_hbm, ew_hbm, zeros_hbm, deg_out,
              col_v, ew_v, degp_v, iidx_v, tmp_v, deg_sh, ssem):
    c = lax.axis_index("c")
    s = lax.axis_index("s")
    wid = c * NS + s
    pltpu.sync_copy(ei_hbm.at[1, wid], col_v)
    pltpu.sync_copy(ew_hbm.at[wid], ew_v)
    rpt = DEG_ROWS // NS
    pltpu.sync_copy(zeros_hbm.at[pl.ds(s * rpt, rpt)],
                    deg_sh.at[pl.ds(s * rpt, rpt)])

    def zb(i, carry):
        degp_v[i, :] = jnp.zeros((LANES,), jnp.float32)
        return carry
    lax.fori_loop(0, DEG_ROWS, zb, 0)
    io = jnp.arange(LANES, dtype=jnp.int32)
    for q in range(DEG_ROWS // CHUNK):
        for k in range(CHUNK // LANES):
            iidx_v[q, pl.ds(k * LANES, LANES)] = io + (q * CHUNK + k * LANES)
    ids1 = jnp.minimum(io + 1, LANES - 1)
    plsc.subcore_barrier()

    def chunk(j, carry):
        for k in range(CHUNK // LANES):
            cols = col_v[j, pl.ds(k * LANES, LANES)]
            ws = ew_v[j, pl.ds(k * LANES, LANES)]
            sk, sv = plsc.sort_key_val(cols, ws)
            ps = plsc.cumsum(sv)
            tmp_v[k, :] = sk
            nextk = plsc.load_gather(tmp_v, [_splat16(k), ids1])
            runend = (sk != nextk) | (io == LANES - 1)
            m2 = runend & (io < LANES - 1)
            r1 = lax.shift_right_logical(sk, 4)
            l1 = lax.bitwise_and(sk, LANES - 1)
            plsc.addupdate_scatter(degp_v, [r1, l1], ps, mask=runend)
            r2 = lax.shift_right_logical(nextk, 4)
            l2 = lax.bitwise_and(nextk, LANES - 1)
            plsc.addupdate_scatter(degp_v, [r2, l2], -ps, mask=m2)
        return carry

    lax.fori_loop(0, EPW_CHUNKS, chunk, 0)

    for q in range(DEG_ROWS // CHUNK):
        pltpu.sync_copy(degp_v.at[pl.ds(q * CHUNK, CHUNK)],
                        deg_sh.at[iidx_v.at[q]], add=True)
    plsc.subcore_barrier()
    pltpu.sync_copy(deg_sh.at[pl.ds(s * rpt, rpt)],
                    deg_out.at[c, pl.ds(s * rpt, rpt)])


@functools.cache
def _make_deg_kernel():
    return pl.kernel(
        _deg_body,
        out_type=jax.ShapeDtypeStruct((NC, DEG_ROWS, LANES), jnp.float32),
        mesh=plsc.VectorSubcoreMesh(core_axis_name="c", subcore_axis_name="s"),
        scratch_types=[
            pltpu.VMEM((EPW_CHUNKS + 1, CHUNK), jnp.int32),
            pltpu.VMEM((EPW_CHUNKS + 1, CHUNK), jnp.float32),
            pltpu.VMEM((DEG_ROWS, LANES), jnp.float32),
            pltpu.VMEM((DEG_ROWS // CHUNK, CHUNK), jnp.int32),
            pltpu.VMEM((CHUNK // LANES, LANES), jnp.int32),
            pltpu.VMEM_SHARED((DEG_ROWS, LANES), jnp.float32),
            pltpu.SemaphoreType.DMA,
        ],
        compiler_params=pltpu.CompilerParams(use_tc_tiling_on_sc=False, needs_layout_passes=False),
    )


def _prep_body(degmat_ref, x_ref, y_ref, dinv_ref):
    deg = degmat_ref[0] + degmat_ref[1] + 2.0
    dinv = lax.rsqrt(deg)
    dinv_ref[...] = dinv
    y_ref[...] = x_ref[...] * dinv


def _run_prep(degmat, x_pad):
    NBP = 2048
    return pl.pallas_call(
        _prep_body,
        grid=(N_PAD // NBP,),
        in_specs=[
            pl.BlockSpec((NC, NBP, 1), lambda i: (0, i, 0)),
            pl.BlockSpec((NBP, LANES), lambda i: (i, 0)),
        ],
        out_specs=[pl.BlockSpec((NBP, LANES), lambda i: (i, 0)),
                   pl.BlockSpec((NBP, 1), lambda i: (i, 0))],
        out_shape=[jax.ShapeDtypeStruct((N_PAD, LANES), jnp.float32),
                   jax.ShapeDtypeStruct((N_PAD, 1), jnp.float32)],
    )(degmat, x_pad)


def _z_body(ei_hbm, ew_hbm, y_hbm, zeros_hbm, z_out,
            row_v, col_v, ew_v,
            ybuf0, ybuf1, ybuf2, zbuf0, zbuf1, zbuf2,
            y_sh, z_sh, gsem, ssem):
    c = lax.axis_index("c")
    s = lax.axis_index("s")
    wid = c * NS + s
    ybufs = (ybuf0, ybuf1, ybuf2)
    zbufs = (zbuf0, zbuf1, zbuf2)
    pltpu.sync_copy(ei_hbm.at[0, wid], row_v)
    pltpu.sync_copy(ei_hbm.at[1, wid], col_v)
    pltpu.sync_copy(ew_hbm.at[wid], ew_v)
    pltpu.sync_copy(zeros_hbm.at[pl.ds(s * ROWS_PER_TILE, ROWS_PER_TILE)],
                    z_sh.at[pl.ds(s * ROWS_PER_TILE, ROWS_PER_TILE)])
    pltpu.sync_copy(y_hbm.at[pl.ds(s * ROWS_PER_TILE, ROWS_PER_TILE)],
                    y_sh.at[pl.ds(s * ROWS_PER_TILE, ROWS_PER_TILE)])
    plsc.subcore_barrier()

    for r in range(CHUNK):
        zbuf2[r, :] = jnp.zeros((LANES,), jnp.float32)
    pltpu.async_copy(zbuf2, z_sh.at[col_v.at[EPW_CHUNKS]], ssem, add=True)
    pltpu.async_copy(zbuf2, z_sh.at[col_v.at[EPW_CHUNKS]], ssem, add=True)
    pltpu.async_copy(y_sh.at[row_v.at[0]], ybuf0, gsem)

    def trip(p, carry):
        for b in range(3):
            j = 3 * p + b
            nb = (b + 1) % 3
            pltpu.async_copy(y_sh.at[row_v.at[j + 1]], ybufs[nb], gsem)
            pltpu.make_async_copy(zbufs[b], z_sh.at[col_v.at[j]], ssem).wait()
            pltpu.make_async_copy(y_sh.at[row_v.at[j]], ybufs[b], gsem).wait()
            js = _splat16(j)
            for r in range(CHUNK):
                w = plsc.load_gather(ew_v, [js, _splat16(r)])
                zbufs[b][r, :] = ybufs[b][r, :] * w
            pltpu.async_copy(zbufs[b], z_sh.at[col_v.at[j]], ssem, add=True)
        return carry

    lax.fori_loop(0, EPW_CHUNKS // 3, trip, 0)
    pltpu.make_async_copy(zbuf0, z_sh.at[col_v.at[0]], ssem).wait()
    pltpu.make_async_copy(zbuf1, z_sh.at[col_v.at[0]], ssem).wait()
    pltpu.make_async_copy(y_sh.at[row_v.at[0]], ybuf0, gsem).wait()
    plsc.subcore_barrier()
    pltpu.sync_copy(z_sh.at[pl.ds(s * ROWS_PER_TILE, ROWS_PER_TILE)],
                    z_out.at[c, pl.ds(s * ROWS_PER_TILE, ROWS_PER_TILE)])


@functools.cache
def _make_z_kernel():
    return pl.kernel(
        _z_body,
        out_type=jax.ShapeDtypeStruct((NC, N_PAD, LANES), jnp.float32),
        mesh=plsc.VectorSubcoreMesh(core_axis_name="c", subcore_axis_name="s"),
        scratch_types=(
            [pltpu.VMEM((EPW_CHUNKS + 1, CHUNK), jnp.int32)] * 2
            + [pltpu.VMEM((EPW_CHUNKS + 1, CHUNK), jnp.float32)]
            + [pltpu.VMEM((CHUNK, LANES), jnp.float32)] * 6
            + [pltpu.VMEM_SHARED((N_PAD, LANES), jnp.float32)] * 2
            + [pltpu.SemaphoreType.DMA, pltpu.SemaphoreType.DMA]
        ),
        compiler_params=pltpu.CompilerParams(use_tc_tiling_on_sc=False, needs_layout_passes=False),
    )


def _gru_body(z_ref, y_ref, dinv_ref, wc_ref, bc_ref, w1_ref, b1_ref,
              w2_ref, b2_ref, out_ref):
    dinv = dinv_ref[...]
    s_all = dinv * (z_ref[0] + z_ref[1] + 2.0 * y_ref[...])
    wc = wc_ref[...]
    bc = bc_ref[...]
    b1 = b1_ref[...]
    b2 = b2_ref[...]
    w1 = w1_ref[...]
    w2 = w2_ref[...]
    h = jnp.zeros((NB, HID), jnp.float32)
    for t in range(PRE_LEN):
        st = s_all[:, t:t + 1]
        f = jax.nn.sigmoid(st * wc + bc)
        cat1 = jnp.concatenate([f, h], axis=1)
        ru = jax.nn.sigmoid(
            jnp.dot(cat1, w1, preferred_element_type=jnp.float32) + b1)
        r = ru[:, :HID]
        u = ru[:, HID:]
        cat2 = jnp.concatenate([f, r * h], axis=1)
        cnew = jnp.tanh(
            jnp.dot(cat2, w2, preferred_element_type=jnp.float32) + b2)
        h = u * h + (1.0 - u) * cnew
    out_ref[...] = h


def _run_gru(zmat, y2, dinv2, Wc, bc, W1, b1, W2, b2):
    grid = (N_BLOCKS,)
    return pl.pallas_call(
        _gru_body,
        grid=grid,
        in_specs=[
            pl.BlockSpec((NC, NB, LANES), lambda i: (0, i, 0)),
            pl.BlockSpec((NB, LANES), lambda i: (i, 0)),
            pl.BlockSpec((NB, 1), lambda i: (i, 0)),
            pl.BlockSpec((1, HID), lambda i: (0, 0)),
            pl.BlockSpec((1, HID), lambda i: (0, 0)),
            pl.BlockSpec((2 * HID, 2 * HID), lambda i: (0, 0)),
            pl.BlockSpec((1, 2 * HID), lambda i: (0, 0)),
            pl.BlockSpec((2 * HID, HID), lambda i: (0, 0)),
            pl.BlockSpec((1, HID), lambda i: (0, 0)),
        ],
        out_specs=pl.BlockSpec((NB, HID), lambda i: (i, 0)),
        out_shape=jax.ShapeDtypeStruct((N_NODES, HID), jnp.float32),
    )(zmat, y2, dinv2, Wc, bc, W1, b1, W2, b2)


def kernel(x, edge_index, edge_weight, Wc, bc, W1, b1, W2, b2):
    E = edge_weight.shape[0]
    row = edge_index[0].astype(jnp.int32)
    col = edge_index[1].astype(jnp.int32)
    ew = edge_weight.astype(jnp.float32)

    pad_e = E_PAD - E
    ei4 = jnp.pad(
        jnp.pad(edge_index.astype(jnp.int32), ((0, 0), (0, pad_e))).reshape(
            2, NW, EPW_CHUNKS, CHUNK),
        ((0, 0), (0, 0), (0, 1), (0, 0)))
    ew3 = jnp.pad(jnp.pad(ew, (0, pad_e)).reshape(NW, EPW_CHUNKS, CHUNK),
                  ((0, 0), (0, 1), (0, 0)))

    x_pad = jnp.pad(x, ((0, N_PAD - N_NODES), (0, LANES - PRE_LEN)))
    zeros_pad = jnp.zeros((N_PAD, LANES), jnp.float32)

    degmat = _make_deg_kernel()(ei4, ew3, zeros_pad)
    degmat = degmat.reshape(NC, N_PAD, 1)

    y2, dinv2 = _run_prep(degmat, x_pad)

    zmat = _make_z_kernel()(ei4, ew3, y2, zeros_pad)

    return _run_gru(zmat, y2, dinv2, Wc, bc.reshape(1, HID), W1,
                    b1.reshape(1, 2 * HID), W2, b2.reshape(1, HID))


# speedup vs baseline: 2.4553x; 1.0056x over previous
import functools

import jax
import jax.numpy as jnp
from jax import lax
from jax.experimental import pallas as pl
from jax.experimental.pallas import tpu as pltpu
from jax.experimental.pallas import tpu_sc as plsc

N_NODES = 10000
HID = 128
PRE_LEN = 12
LANES = 16
N_PAD = 10240
NC = 2
NS = 16
NW = NC * NS
CHUNK = 128
EPW_CHUNKS = 81
E_PAD = NW * EPW_CHUNKS * CHUNK
ROWS_PER_TILE = N_PAD // NS

NB = 2048
N_BLOCKS = N_PAD // NB


def _splat16(v):
    return jnp.zeros((LANES,), jnp.int32) + v


DEG_ROWS = N_PAD // LANES


def _deg_body(ei_hbm, ew_hbm, zeros_hbm, deg_out,
              col_v, ew_v, degp_v, iidx_v, tmp_v, deg_sh, ssem):
    c = lax.axis_index("c")
    s = lax.axis_index("s")
    wid = c * NS + s
    pltpu.sync_copy(ei_hbm.at[1, wid], col_v)
    pltpu.sync_copy(ew_hbm.at[wid], ew_v)
    rpt = DEG_ROWS // NS
    pltpu.sync_copy(zeros_hbm.at[pl.ds(s * rpt, rpt)],
                    deg_sh.at[pl.ds(s * rpt, rpt)])

    def zb(i, carry):
        degp_v[i, :] = jnp.zeros((LANES,), jnp.float32)
        return carry
    lax.fori_loop(0, DEG_ROWS, zb, 0)
    io = jnp.arange(LANES, dtype=jnp.int32)
    for q in range(DEG_ROWS // CHUNK):
        for k in range(CHUNK // LANES):
            iidx_v[q, pl.ds(k * LANES, LANES)] = io + (q * CHUNK + k * LANES)
    ids1 = jnp.minimum(io + 1, LANES - 1)
    plsc.subcore_barrier()

    def chunk(j, carry):
        for k in range(CHUNK // LANES):
            cols = col_v[j, pl.ds(k * LANES, LANES)]
            ws = ew_v[j, pl.ds(k * LANES, LANES)]
            sk, sv = plsc.sort_key_val(cols, ws)
            ps = plsc.cumsum(sv)
            tmp_v[k, :] = sk
            nextk = plsc.load_gather(tmp_v, [_splat16(k), ids1])
            runend = (sk != nextk) | (io == LANES - 1)
            m2 = runend & (io < LANES - 1)
            r1 = lax.shift_right_logical(sk, 4)
            l1 = lax.bitwise_and(sk, LANES - 1)
            plsc.addupdate_scatter(degp_v, [r1, l1], ps, mask=runend)
            r2 = lax.shift_right_logical(nextk, 4)
            l2 = lax.bitwise_and(nextk, LANES - 1)
            plsc.addupdate_scatter(degp_v, [r2, l2], -ps, mask=m2)
        return carry

    lax.fori_loop(0, EPW_CHUNKS, chunk, 0)

    for q in range(DEG_ROWS // CHUNK):
        pltpu.sync_copy(degp_v.at[pl.ds(q * CHUNK, CHUNK)],
                        deg_sh.at[iidx_v.at[q]], add=True)
    plsc.subcore_barrier()
    pltpu.sync_copy(deg_sh.at[pl.ds(s * rpt, rpt)],
                    deg_out.at[c, pl.ds(s * rpt, rpt)])


@functools.cache
def _make_deg_kernel():
    return pl.kernel(
        _deg_body,
        out_type=jax.ShapeDtypeStruct((NC, DEG_ROWS, LANES), jnp.float32),
        mesh=plsc.VectorSubcoreMesh(core_axis_name="c", subcore_axis_name="s"),
        scratch_types=[
            pltpu.VMEM((EPW_CHUNKS + 1, CHUNK), jnp.int32),
            pltpu.VMEM((EPW_CHUNKS + 1, CHUNK), jnp.float32),
            pltpu.VMEM((DEG_ROWS, LANES), jnp.float32),
            pltpu.VMEM((DEG_ROWS // CHUNK, CHUNK), jnp.int32),
            pltpu.VMEM((CHUNK // LANES, LANES), jnp.int32),
            pltpu.VMEM_SHARED((DEG_ROWS, LANES), jnp.float32),
            pltpu.SemaphoreType.DMA,
        ],
        compiler_params=pltpu.CompilerParams(use_tc_tiling_on_sc=False, needs_layout_passes=False),
    )


def _prep_body(degmat_ref, x_ref, y_ref, dinv_ref):
    deg = degmat_ref[0] + degmat_ref[1] + 2.0
    dinv = lax.rsqrt(deg)
    dinv_ref[...] = dinv
    y_ref[...] = x_ref[...] * dinv


def _run_prep(degmat, x_pad):
    NBP = 2048
    return pl.pallas_call(
        _prep_body,
        grid=(N_PAD // NBP,),
        in_specs=[
            pl.BlockSpec((NC, NBP, 1), lambda i: (0, i, 0)),
            pl.BlockSpec((NBP, LANES), lambda i: (i, 0)),
        ],
        out_specs=[pl.BlockSpec((NBP, LANES), lambda i: (i, 0)),
                   pl.BlockSpec((NBP, 1), lambda i: (i, 0))],
        out_shape=[jax.ShapeDtypeStruct((N_PAD, LANES), jnp.float32),
                   jax.ShapeDtypeStruct((N_PAD, 1), jnp.float32)],
    )(degmat, x_pad)


def _z_body(ei_hbm, ew_hbm, y_hbm, zeros_hbm, z_out,
            row_v, col_v, ew_v,
            ybuf0, ybuf1, ybuf2, zbuf0, zbuf1, zbuf2,
            y_sh, z_sh, gsem, ssem):
    c = lax.axis_index("c")
    s = lax.axis_index("s")
    wid = c * NS + s
    ybufs = (ybuf0, ybuf1, ybuf2)
    zbufs = (zbuf0, zbuf1, zbuf2)
    pltpu.sync_copy(ei_hbm.at[0, wid], row_v)
    pltpu.sync_copy(ei_hbm.at[1, wid], col_v)
    pltpu.sync_copy(ew_hbm.at[wid], ew_v)
    pltpu.sync_copy(zeros_hbm.at[pl.ds(s * ROWS_PER_TILE, ROWS_PER_TILE)],
                    z_sh.at[pl.ds(s * ROWS_PER_TILE, ROWS_PER_TILE)])
    pltpu.sync_copy(y_hbm.at[pl.ds(s * ROWS_PER_TILE, ROWS_PER_TILE)],
                    y_sh.at[pl.ds(s * ROWS_PER_TILE, ROWS_PER_TILE)])
    plsc.subcore_barrier()

    for r in range(CHUNK):
        zbuf2[r, :] = jnp.zeros((LANES,), jnp.float32)
    pltpu.async_copy(zbuf2, z_sh.at[col_v.at[EPW_CHUNKS]], ssem, add=True)
    pltpu.async_copy(zbuf2, z_sh.at[col_v.at[EPW_CHUNKS]], ssem, add=True)
    pltpu.async_copy(y_sh.at[row_v.at[0]], ybuf0, gsem)

    def trip(p, carry):
        for b in range(3):
            j = 3 * p + b
            nb = (b + 1) % 3
            pltpu.async_copy(y_sh.at[row_v.at[j + 1]], ybufs[nb], gsem)
            pltpu.make_async_copy(zbufs[b], z_sh.at[col_v.at[j]], ssem).wait()
            pltpu.make_async_copy(y_sh.at[row_v.at[j]], ybufs[b], gsem).wait()
            js = _splat16(j)
            for r in range(CHUNK):
                w = plsc.load_gather(ew_v, [js, _splat16(r)])
                zbufs[b][r, :] = ybufs[b][r, :] * w
            pltpu.async_copy(zbufs[b], z_sh.at[col_v.at[j]], ssem, add=True)
        return carry

    lax.fori_loop(0, EPW_CHUNKS // 3, trip, 0)
    pltpu.make_async_copy(zbuf0, z_sh.at[col_v.at[0]], ssem).wait()
    pltpu.make_async_copy(zbuf1, z_sh.at[col_v.at[0]], ssem).wait()
    pltpu.make_async_copy(y_sh.at[row_v.at[0]], ybuf0, gsem).wait()
    plsc.subcore_barrier()
    pltpu.sync_copy(z_sh.at[pl.ds(s * ROWS_PER_TILE, ROWS_PER_TILE)],
                    z_out.at[c, pl.ds(s * ROWS_PER_TILE, ROWS_PER_TILE)])


@functools.cache
def _make_z_kernel():
    return pl.kernel(
        _z_body,
        out_type=jax.ShapeDtypeStruct((NC, N_PAD, LANES), jnp.float32),
        mesh=plsc.VectorSubcoreMesh(core_axis_name="c", subcore_axis_name="s"),
        scratch_types=(
            [pltpu.VMEM((EPW_CHUNKS + 1, CHUNK), jnp.int32)] * 2
            + [pltpu.VMEM((EPW_CHUNKS + 1, CHUNK), jnp.float32)]
            + [pltpu.VMEM((CHUNK, LANES), jnp.float32)] * 6
            + [pltpu.VMEM_SHARED((N_PAD, LANES), jnp.float32)] * 2
            + [pltpu.SemaphoreType.DMA, pltpu.SemaphoreType.DMA]
        ),
        compiler_params=pltpu.CompilerParams(use_tc_tiling_on_sc=False, needs_layout_passes=False),
    )


def _gru_body(z_ref, y_ref, dinv_ref, wc_ref, bc_ref, w1_ref, b1_ref,
              w2_ref, b2_ref, out_ref):
    dinv = dinv_ref[...]
    s_all = dinv * (z_ref[0] + z_ref[1] + 2.0 * y_ref[...])
    wc = wc_ref[...]
    bc = bc_ref[...]
    b1 = b1_ref[...]
    b2 = b2_ref[...]
    w1 = w1_ref[...]
    w2 = w2_ref[...]
    h = jnp.zeros((NB, HID), jnp.float32)
    for t in range(PRE_LEN):
        st = s_all[:, t:t + 1]
        f = jax.nn.sigmoid(st * wc + bc)
        cat1 = jnp.concatenate([f, h], axis=1)
        ru = jax.nn.sigmoid(
            jnp.dot(cat1, w1, preferred_element_type=jnp.float32) + b1)
        r = ru[:, :HID]
        u = ru[:, HID:]
        cat2 = jnp.concatenate([f, r * h], axis=1)
        cnew = jnp.tanh(
            jnp.dot(cat2, w2, preferred_element_type=jnp.float32) + b2)
        h = u * h + (1.0 - u) * cnew
    out_ref[...] = h


def _run_gru(zmat, y2, dinv2, Wc, bc, W1, b1, W2, b2):
    grid = (N_BLOCKS,)
    return pl.pallas_call(
        _gru_body,
        grid=grid,
        in_specs=[
            pl.BlockSpec((NC, NB, LANES), lambda i: (0, i, 0)),
            pl.BlockSpec((NB, LANES), lambda i: (i, 0)),
            pl.BlockSpec((NB, 1), lambda i: (i, 0)),
            pl.BlockSpec((1, HID), lambda i: (0, 0)),
            pl.BlockSpec((1, HID), lambda i: (0, 0)),
            pl.BlockSpec((2 * HID, 2 * HID), lambda i: (0, 0)),
            pl.BlockSpec((1, 2 * HID), lambda i: (0, 0)),
            pl.BlockSpec((2 * HID, HID), lambda i: (0, 0)),
            pl.BlockSpec((1, HID), lambda i: (0, 0)),
        ],
        out_specs=pl.BlockSpec((NB, HID), lambda i: (i, 0)),
        out_shape=jax.ShapeDtypeStruct((N_NODES, HID), jnp.float32),
    )(zmat, y2, dinv2, Wc, bc, W1, b1, W2, b2)


def kernel(x, edge_index, edge_weight, Wc, bc, W1, b1, W2, b2):
    E = edge_weight.shape[0]
    row = edge_index[0].astype(jnp.int32)
    col = edge_index[1].astype(jnp.int32)
    ew = edge_weight.astype(jnp.float32)

    pad_e = E_PAD - E
    ei4 = jnp.pad(
        jnp.pad(edge_index.astype(jnp.int32), ((0, 0), (0, pad_e))).reshape(
            2, NW, EPW_CHUNKS, CHUNK),
        ((0, 0), (0, 0), (0, 1), (0, 0)))
    ew3 = jnp.pad(jnp.pad(ew, (0, pad_e)).reshape(NW, EPW_CHUNKS, CHUNK),
                  ((0, 0), (0, 1), (0, 0)))

    x_pad = jnp.pad(x, ((0, N_PAD - N_NODES), (0, LANES - PRE_LEN)))
    zeros_pad = jnp.zeros((N_PAD, LANES), jnp.float32)

    degmat = _make_deg_kernel()(ei4, ew3, zeros_pad)
    degmat = degmat.reshape(NC, N_PAD, 1)

    y2, dinv2 = _run_prep(degmat, x_pad)

    zmat = _make_z_kernel()(ei4, ew3, y2, zeros_pad)

    return _run_gru(zmat, y2, dinv2, Wc, bc.reshape(1, HID), W1,
                    b1.reshape(1, 2 * HID), W2, b2.reshape(1, HID))
